# Initial kernel scaffold; baseline (speedup 1.0000x reference)
import functools

import jax
import jax.numpy as jnp
from jax import lax
from jax.experimental import pallas as pl
from jax.experimental.pallas import tpu as pltpu
from jax.experimental.pallas import tpu_sc as plsc

VOCAB_ROWS = 100000
EMB = 64
B = 4096
QLEN = 20
NDOCS = 8
DLEN = 50

NC = 2
NS = 16
NW = NC * NS

Q_SEGS_W = B // NW
Q_SEGS_CHUNK = 4
Q_CHUNK_ROWS = Q_SEGS_CHUNK * QLEN
Q_CHUNKS = Q_SEGS_W // Q_SEGS_CHUNK

D_SEGS = B * NDOCS
D_SEGS_W = D_SEGS // NW
D_SEGS_CHUNK = 2
D_CHUNK_ROWS_REAL = D_SEGS_CHUNK * DLEN
D_CHUNK_PAD = 104
D_CHUNKS = D_SEGS_W // D_SEGS_CHUNK
D_FLUSH_CHUNKS = 64
D_BLOCKS = D_CHUNKS // D_FLUSH_CHUNKS


def _sc_pool(table, qidx, didx):
    mesh = plsc.VectorSubcoreMesh(
        core_axis_name="c", subcore_axis_name="s",
        num_cores=NC, num_subcores=NS)

    @functools.partial(
        pl.kernel,
        out_type=[
            jax.ShapeDtypeStruct((B, EMB), jnp.float32),
            jax.ShapeDtypeStruct((D_SEGS, EMB), jnp.float32),
        ],
        mesh=mesh,
        scratch_types=[
            pltpu.VMEM((Q_SEGS_W * QLEN,), jnp.int32),
            pltpu.VMEM((D_CHUNKS * D_CHUNK_PAD,), jnp.int32),
            pltpu.VMEM((Q_CHUNK_ROWS, EMB), jnp.float32),
            pltpu.VMEM((D_CHUNK_PAD, EMB), jnp.float32),
            pltpu.VMEM((Q_SEGS_W, EMB), jnp.float32),
            pltpu.VMEM((D_SEGS_CHUNK * D_FLUSH_CHUNKS, EMB), jnp.float32),
            pltpu.SemaphoreType.DMA,
        ],
    )
    def pool(table_hbm, qidx_hbm, didx_hbm, qsum_hbm, dsum_hbm,
             qidx_v, didx_v, qrows_v, drows_v, qstage_v, dstage_v, sem):
        wid = lax.axis_index("s") * NC + lax.axis_index("c")

        pltpu.sync_copy(qidx_hbm.at[wid], qidx_v)
        pltpu.sync_copy(didx_hbm.at[wid], didx_v)

        def q_chunk(c, carry):
            idx = qidx_v.at[pl.ds(c * Q_CHUNK_ROWS, Q_CHUNK_ROWS)]
            pltpu.async_copy(table_hbm.at[idx], qrows_v, sem).wait()
            for s in range(Q_SEGS_CHUNK):
                for j in range(EMB // 16):
                    acc = qrows_v[s * QLEN, pl.ds(j * 16, 16)]
                    for r in range(1, QLEN):
                        acc = acc + qrows_v[s * QLEN + r, pl.ds(j * 16, 16)]
                    qstage_v[c * Q_SEGS_CHUNK + s, pl.ds(j * 16, 16)] = acc
            return carry
        lax.fori_loop(0, Q_CHUNKS, q_chunk, 0)
        pltpu.sync_copy(qstage_v, qsum_hbm.at[pl.ds(wid * Q_SEGS_W, Q_SEGS_W)])

        def d_block(blk, carry):
            def d_chunk(cc, inner):
                c = blk * D_FLUSH_CHUNKS + cc
                idx = didx_v.at[pl.ds(c * D_CHUNK_PAD, D_CHUNK_PAD)]
                pltpu.async_copy(table_hbm.at[idx], drows_v, sem).wait()
                for s in range(D_SEGS_CHUNK):
                    for j in range(EMB // 16):
                        acc = drows_v[s * DLEN, pl.ds(j * 16, 16)]
                        for r in range(1, DLEN):
                            acc = acc + drows_v[s * DLEN + r, pl.ds(j * 16, 16)]
                        dstage_v[cc * D_SEGS_CHUNK + s, pl.ds(j * 16, 16)] = acc
                return inner
            lax.fori_loop(0, D_FLUSH_CHUNKS, d_chunk, 0)
            nsegs = D_SEGS_CHUNK * D_FLUSH_CHUNKS
            pltpu.sync_copy(
                dstage_v,
                dsum_hbm.at[pl.ds(wid * D_SEGS_W + blk * nsegs, nsegs)])
            return carry
        lax.fori_loop(0, D_BLOCKS, d_block, 0)

    return pool(table, qidx, didx)


def _cosine_tc(qsum, dsum):
    BLK = 256

    def body(q_ref, d_ref, o_ref):
        q = q_ref[...] * (1.0 / QLEN)
        qn = jnp.maximum(
            jnp.sqrt(jnp.sum(q * q, axis=1, keepdims=True)), 1e-8)
        cols = []
        for dd in range(NDOCS):
            dj = d_ref[:, dd * EMB:(dd + 1) * EMB] * (1.0 / DLEN)
            dot = jnp.sum(q * dj, axis=1, keepdims=True)
            dn = jnp.maximum(
                jnp.sqrt(jnp.sum(dj * dj, axis=1, keepdims=True)), 1e-8)
            cols.append(dot / (qn * dn))
        o_ref[...] = jnp.concatenate(cols, axis=1)

    return pl.pallas_call(
        body,
        grid=(B // BLK,),
        in_specs=[
            pl.BlockSpec((BLK, EMB), lambda i: (i, 0)),
            pl.BlockSpec((BLK, NDOCS * EMB), lambda i: (i, 0)),
        ],
        out_specs=pl.BlockSpec((BLK, NDOCS), lambda i: (i, 0)),
        out_shape=jax.ShapeDtypeStruct((B, NDOCS), jnp.float32),
    )(qsum, dsum)


def kernel(batch_queries, query_len, batch_docs, doc_len, W):
    del query_len, doc_len
    qidx = batch_queries.astype(jnp.int32).reshape(NW, Q_SEGS_W * QLEN)
    d = batch_docs.astype(jnp.int32).reshape(NW, D_CHUNKS, D_CHUNK_ROWS_REAL)
    pad = jnp.zeros((NW, D_CHUNKS, D_CHUNK_PAD - D_CHUNK_ROWS_REAL),
                    jnp.int32)
    didx = jnp.concatenate([d, pad], axis=2).reshape(NW, -1)
    qsum, dsum = _sc_pool(W, qidx, didx)
    scores = _cosine_tc(qsum, dsum.reshape(B, NDOCS * EMB))
    return scores


# speedup vs baseline: 9.9854x; 9.9854x over previous
import functools

import jax
import jax.numpy as jnp
from jax import lax
from jax.experimental import pallas as pl
from jax.experimental.pallas import tpu as pltpu
from jax.experimental.pallas import tpu_sc as plsc

VOCAB_ROWS = 100000
EMB = 64
B = 4096
QLEN = 20
NDOCS = 8
DLEN = 50

NC = 2
NS = 16
NW = NC * NS

Q_SEGS_W = B // NW
Q_SEGS_CHUNK = 4
Q_CHUNK_ROWS = Q_SEGS_CHUNK * QLEN
Q_CHUNKS = Q_SEGS_W // Q_SEGS_CHUNK

D_SEGS = B * NDOCS
D_SEGS_W = D_SEGS // NW
D_SEGS_CHUNK = 2
D_CHUNK_ROWS_REAL = D_SEGS_CHUNK * DLEN
D_CHUNK_PAD = 104
D_CHUNKS = D_SEGS_W // D_SEGS_CHUNK
D_FLUSH_CHUNKS = 64
D_BLOCKS = D_CHUNKS // D_FLUSH_CHUNKS


def _sc_pool(table, qidx, didx):
    mesh = plsc.VectorSubcoreMesh(
        core_axis_name="c", subcore_axis_name="s",
        num_cores=NC, num_subcores=NS)

    @functools.partial(
        pl.kernel,
        out_type=[
            jax.ShapeDtypeStruct((B, EMB), jnp.float32),
            jax.ShapeDtypeStruct((D_SEGS, EMB), jnp.float32),
        ],
        mesh=mesh,
        compiler_params=pltpu.CompilerParams(use_tc_tiling_on_sc=False),
        scratch_types=[
            pltpu.VMEM((Q_SEGS_W * QLEN,), jnp.int32),
            pltpu.VMEM((D_CHUNKS * D_CHUNK_PAD,), jnp.int32),
            pltpu.VMEM((Q_CHUNK_ROWS, EMB), jnp.float32),
            pltpu.VMEM((D_CHUNK_PAD, EMB), jnp.float32),
            pltpu.VMEM((Q_SEGS_W, EMB), jnp.float32),
            pltpu.VMEM((D_SEGS_CHUNK * D_FLUSH_CHUNKS, EMB), jnp.float32),
            pltpu.SemaphoreType.DMA,
        ],
    )
    def pool(table_hbm, qidx_hbm, didx_hbm, qsum_hbm, dsum_hbm,
             qidx_v, didx_v, qrows_v, drows_v, qstage_v, dstage_v, sem):
        wid = lax.axis_index("s") * NC + lax.axis_index("c")

        pltpu.sync_copy(qidx_hbm.at[wid], qidx_v)
        pltpu.sync_copy(didx_hbm.at[wid], didx_v)

        def q_chunk(c, carry):
            idx = qidx_v.at[pl.ds(c * Q_CHUNK_ROWS, Q_CHUNK_ROWS)]
            pltpu.async_copy(table_hbm.at[idx], qrows_v, sem).wait()
            for s in range(Q_SEGS_CHUNK):
                for j in range(EMB // 16):
                    acc = qrows_v[s * QLEN, pl.ds(j * 16, 16)]
                    for r in range(1, QLEN):
                        acc = acc + qrows_v[s * QLEN + r, pl.ds(j * 16, 16)]
                    qstage_v[c * Q_SEGS_CHUNK + s, pl.ds(j * 16, 16)] = acc
            return carry
        lax.fori_loop(0, Q_CHUNKS, q_chunk, 0)
        pltpu.sync_copy(qstage_v, qsum_hbm.at[pl.ds(wid * Q_SEGS_W, Q_SEGS_W)])

        def d_block(blk, carry):
            def d_chunk(cc, inner):
                c = blk * D_FLUSH_CHUNKS + cc
                idx = didx_v.at[pl.ds(c * D_CHUNK_PAD, D_CHUNK_PAD)]
                pltpu.async_copy(table_hbm.at[idx], drows_v, sem).wait()
                for s in range(D_SEGS_CHUNK):
                    for j in range(EMB // 16):
                        acc = drows_v[s * DLEN, pl.ds(j * 16, 16)]
                        for r in range(1, DLEN):
                            acc = acc + drows_v[s * DLEN + r, pl.ds(j * 16, 16)]
                        dstage_v[cc * D_SEGS_CHUNK + s, pl.ds(j * 16, 16)] = acc
                return inner
            lax.fori_loop(0, D_FLUSH_CHUNKS, d_chunk, 0)
            nsegs = D_SEGS_CHUNK * D_FLUSH_CHUNKS
            pltpu.sync_copy(
                dstage_v,
                dsum_hbm.at[pl.ds(wid * D_SEGS_W + blk * nsegs, nsegs)])
            return carry
        lax.fori_loop(0, D_BLOCKS, d_block, 0)

    return pool(table, qidx, didx)


def _cosine_tc(qsum, dsum):
    BLK = 256

    def body(q_ref, d_ref, o_ref):
        q = q_ref[...] * (1.0 / QLEN)
        qn = jnp.maximum(
            jnp.sqrt(jnp.sum(q * q, axis=1, keepdims=True)), 1e-8)
        cols = []
        for dd in range(NDOCS):
            dj = d_ref[:, dd * EMB:(dd + 1) * EMB] * (1.0 / DLEN)
            dot = jnp.sum(q * dj, axis=1, keepdims=True)
            dn = jnp.maximum(
                jnp.sqrt(jnp.sum(dj * dj, axis=1, keepdims=True)), 1e-8)
            cols.append(dot / (qn * dn))
        o_ref[...] = jnp.concatenate(cols, axis=1)

    return pl.pallas_call(
        body,
        grid=(B // BLK,),
        in_specs=[
            pl.BlockSpec((BLK, EMB), lambda i: (i, 0)),
            pl.BlockSpec((BLK, NDOCS * EMB), lambda i: (i, 0)),
        ],
        out_specs=pl.BlockSpec((BLK, NDOCS), lambda i: (i, 0)),
        out_shape=jax.ShapeDtypeStruct((B, NDOCS), jnp.float32),
    )(qsum, dsum)


def kernel(batch_queries, query_len, batch_docs, doc_len, W):
    del query_len, doc_len
    qidx = batch_queries.astype(jnp.int32).reshape(NW, Q_SEGS_W * QLEN)
    d = batch_docs.astype(jnp.int32).reshape(NW, D_CHUNKS, D_CHUNK_ROWS_REAL)
    pad = jnp.zeros((NW, D_CHUNKS, D_CHUNK_PAD - D_CHUNK_ROWS_REAL),
                    jnp.int32)
    didx = jnp.concatenate([d, pad], axis=2).reshape(NW, -1)
    qsum, dsum = _sc_pool(W, qidx, didx)
    scores = _cosine_tc(qsum, dsum.reshape(B, NDOCS * EMB))
    return scores


# speedup vs baseline: 10.0579x; 1.0073x over previous
import functools

import jax
import jax.numpy as jnp
from jax import lax
from jax.experimental import pallas as pl
from jax.experimental.pallas import tpu as pltpu
from jax.experimental.pallas import tpu_sc as plsc

VOCAB_ROWS = 100000
EMB = 64
B = 4096
QLEN = 20
NDOCS = 8
DLEN = 50

NC = 2
NS = 16
NW = NC * NS

Q_SEGS_W = B // NW
Q_SEGS_CHUNK = 4
Q_CHUNK_ROWS = Q_SEGS_CHUNK * QLEN
Q_CHUNKS = Q_SEGS_W // Q_SEGS_CHUNK

D_SEGS = B * NDOCS
D_SEGS_W = D_SEGS // NW
D_SEGS_CHUNK = 2
D_CHUNK_ROWS_REAL = D_SEGS_CHUNK * DLEN
D_CHUNK_PAD = 104
D_CHUNKS = D_SEGS_W // D_SEGS_CHUNK
D_FLUSH_CHUNKS = 64
D_BLOCKS = D_CHUNKS // D_FLUSH_CHUNKS


def _sc_pool(table, qidx, didx):
    mesh = plsc.VectorSubcoreMesh(
        core_axis_name="c", subcore_axis_name="s",
        num_cores=NC, num_subcores=NS)

    NBUF_D = 4
    NBUF_Q = 2
    D_GROUPS = D_FLUSH_CHUNKS // NBUF_D
    Q_GROUPS = Q_CHUNKS // NBUF_Q

    @functools.partial(
        pl.kernel,
        out_type=[
            jax.ShapeDtypeStruct((B, EMB), jnp.float32),
            jax.ShapeDtypeStruct((D_SEGS, EMB), jnp.float32),
        ],
        mesh=mesh,
        compiler_params=pltpu.CompilerParams(use_tc_tiling_on_sc=False),
        scratch_types=[
            pltpu.VMEM((Q_SEGS_W * QLEN,), jnp.int32),
            pltpu.VMEM((D_CHUNKS * D_CHUNK_PAD,), jnp.int32),
            [pltpu.VMEM((Q_CHUNK_ROWS, EMB), jnp.float32)] * NBUF_Q,
            [pltpu.VMEM((D_CHUNK_PAD, EMB), jnp.float32)] * NBUF_D,
            pltpu.VMEM((Q_SEGS_W, EMB), jnp.float32),
            pltpu.VMEM((D_SEGS_CHUNK * D_FLUSH_CHUNKS, EMB), jnp.float32),
            [pltpu.SemaphoreType.DMA] * NBUF_Q,
            [pltpu.SemaphoreType.DMA] * NBUF_D,
        ],
    )
    def pool(table_hbm, qidx_hbm, didx_hbm, qsum_hbm, dsum_hbm,
             qidx_v, didx_v, qbufs, dbufs, qstage_v, dstage_v, qsems, dsems):
        wid = lax.axis_index("s") * NC + lax.axis_index("c")

        pltpu.sync_copy(qidx_hbm.at[wid], qidx_v)
        pltpu.sync_copy(didx_hbm.at[wid], didx_v)

        def q_issue(c, b):
            start = pl.multiple_of(c * Q_CHUNK_ROWS, 8)
            idx = qidx_v.at[pl.ds(start, Q_CHUNK_ROWS)]
            pltpu.async_copy(table_hbm.at[idx], qbufs[b], qsems[b])

        def d_issue(c, b):
            start = pl.multiple_of(c * D_CHUNK_PAD, 8)
            idx = didx_v.at[pl.ds(start, D_CHUNK_PAD)]
            pltpu.async_copy(table_hbm.at[idx], dbufs[b], dsems[b])

        def reduce_seg(buf, row0, seg_len, stage, out_row):
            half = seg_len // 2
            for j in range(EMB // 16):
                lanes = pl.ds(j * 16, 16)
                a = buf[row0, lanes]
                for r in range(1, half):
                    a = a + buf[row0 + r, lanes]
                b = buf[row0 + half, lanes]
                for r in range(half + 1, seg_len):
                    b = b + buf[row0 + r, lanes]
                stage[out_row, lanes] = a + b

        for b in range(NBUF_Q):
            q_issue(b, b)

        def q_group(g, carry):
            for b in range(NBUF_Q):
                c = g * NBUF_Q + b
                pltpu.make_async_copy(
                    table_hbm.at[qidx_v.at[pl.ds(0, Q_CHUNK_ROWS)]],
                    qbufs[b], qsems[b]).wait()
                for s in range(Q_SEGS_CHUNK):
                    reduce_seg(qbufs[b], s * QLEN, QLEN, qstage_v,
                               c * Q_SEGS_CHUNK + s)
                nxt = c + NBUF_Q
                @pl.when(nxt < Q_CHUNKS)
                def _():
                    q_issue(nxt, b)
            return carry
        lax.fori_loop(0, Q_GROUPS, q_group, 0)
        pltpu.sync_copy(qstage_v, qsum_hbm.at[pl.ds(wid * Q_SEGS_W, Q_SEGS_W)])

        for b in range(NBUF_D):
            d_issue(b, b)

        def d_block(blk, carry):
            def d_group(g, inner):
                for b in range(NBUF_D):
                    local = g * NBUF_D + b
                    c = blk * D_FLUSH_CHUNKS + local
                    pltpu.make_async_copy(
                        table_hbm.at[didx_v.at[pl.ds(0, D_CHUNK_PAD)]],
                        dbufs[b], dsems[b]).wait()
                    for s in range(D_SEGS_CHUNK):
                        reduce_seg(dbufs[b], s * DLEN, DLEN, dstage_v,
                                   local * D_SEGS_CHUNK + s)
                    nxt = c + NBUF_D
                    @pl.when(nxt < D_CHUNKS)
                    def _():
                        d_issue(nxt, b)
                return inner
            lax.fori_loop(0, D_GROUPS, d_group, 0)
            nsegs = D_SEGS_CHUNK * D_FLUSH_CHUNKS
            pltpu.sync_copy(
                dstage_v,
                dsum_hbm.at[pl.ds(wid * D_SEGS_W + blk * nsegs, nsegs)])
            return carry
        lax.fori_loop(0, D_BLOCKS, d_block, 0)

    return pool(table, qidx, didx)


def _cosine_tc(qsum, dsum):
    BLK = 256

    def body(q_ref, d_ref, o_ref):
        q = q_ref[...] * (1.0 / QLEN)
        qn = jnp.maximum(
            jnp.sqrt(jnp.sum(q * q, axis=1, keepdims=True)), 1e-8)
        cols = []
        for dd in range(NDOCS):
            dj = d_ref[:, dd * EMB:(dd + 1) * EMB] * (1.0 / DLEN)
            dot = jnp.sum(q * dj, axis=1, keepdims=True)
            dn = jnp.maximum(
                jnp.sqrt(jnp.sum(dj * dj, axis=1, keepdims=True)), 1e-8)
            cols.append(dot / (qn * dn))
        o_ref[...] = jnp.concatenate(cols, axis=1)

    return pl.pallas_call(
        body,
        grid=(B // BLK,),
        in_specs=[
            pl.BlockSpec((BLK, EMB), lambda i: (i, 0)),
            pl.BlockSpec((BLK, NDOCS * EMB), lambda i: (i, 0)),
        ],
        out_specs=pl.BlockSpec((BLK, NDOCS), lambda i: (i, 0)),
        out_shape=jax.ShapeDtypeStruct((B, NDOCS), jnp.float32),
    )(qsum, dsum)


def kernel(batch_queries, query_len, batch_docs, doc_len, W):
    del query_len, doc_len
    qidx = batch_queries.astype(jnp.int32).reshape(NW, Q_SEGS_W * QLEN)
    d = batch_docs.astype(jnp.int32).reshape(NW, D_CHUNKS, D_CHUNK_ROWS_REAL)
    pad = jnp.zeros((NW, D_CHUNKS, D_CHUNK_PAD - D_CHUNK_ROWS_REAL),
                    jnp.int32)
    didx = jnp.concatenate([d, pad], axis=2).reshape(NW, -1)
    qsum, dsum = _sc_pool(W, qidx, didx)
    scores = _cosine_tc(qsum, dsum.reshape(B, NDOCS * EMB))
    return scores


# speedup vs baseline: 17.3957x; 1.7295x over previous
import functools

import jax
import jax.numpy as jnp
from jax import lax
from jax.experimental import pallas as pl
from jax.experimental.pallas import tpu as pltpu
from jax.experimental.pallas import tpu_sc as plsc

VOCAB_ROWS = 100000
EMB = 64
B = 4096
QLEN = 20
NDOCS = 8
DLEN = 50

NC = 2
NS = 16
NW = NC * NS

Q_SEGS_W = B // NW
Q_SEGS_CHUNK = 4
Q_CHUNK_ROWS = Q_SEGS_CHUNK * QLEN
Q_CHUNKS = Q_SEGS_W // Q_SEGS_CHUNK

D_SEGS = B * NDOCS
D_SEGS_W = D_SEGS // NW
D_SEGS_CHUNK = 2
D_CHUNK_ROWS_REAL = D_SEGS_CHUNK * DLEN
D_CHUNK_PAD = 104
D_CHUNKS = D_SEGS_W // D_SEGS_CHUNK
D_FLUSH_CHUNKS = 64
D_BLOCKS = D_CHUNKS // D_FLUSH_CHUNKS


def _sc_pool(table, qidx, didx):
    mesh = plsc.VectorSubcoreMesh(
        core_axis_name="c", subcore_axis_name="s",
        num_cores=NC, num_subcores=NS)

    NBUF_D = 4
    NBUF_Q = 2
    D_GROUPS = D_FLUSH_CHUNKS // NBUF_D
    Q_GROUPS = Q_CHUNKS // NBUF_Q

    @functools.partial(
        pl.kernel,
        out_type=[
            jax.ShapeDtypeStruct((B, EMB), jnp.float32),
            jax.ShapeDtypeStruct((D_SEGS, EMB), jnp.float32),
        ],
        mesh=mesh,
        compiler_params=pltpu.CompilerParams(
            use_tc_tiling_on_sc=False, needs_layout_passes=False),
        scratch_types=[
            pltpu.VMEM((Q_SEGS_W * QLEN,), jnp.int32),
            pltpu.VMEM((D_CHUNKS * D_CHUNK_PAD,), jnp.int32),
            [pltpu.VMEM((Q_CHUNK_ROWS, EMB), jnp.bfloat16)] * NBUF_Q,
            [pltpu.VMEM((D_CHUNK_PAD, EMB), jnp.bfloat16)] * NBUF_D,
            pltpu.VMEM((Q_SEGS_W, EMB), jnp.float32),
            pltpu.VMEM((D_SEGS_CHUNK * D_FLUSH_CHUNKS, EMB), jnp.float32),
            [pltpu.SemaphoreType.DMA] * NBUF_Q,
            [pltpu.SemaphoreType.DMA] * NBUF_D,
        ],
    )
    def pool(table_hbm, qidx_hbm, didx_hbm, qsum_hbm, dsum_hbm,
             qidx_v, didx_v, qbufs, dbufs, qstage_v, dstage_v, qsems, dsems):
        wid = lax.axis_index("s") * NC + lax.axis_index("c")

        pltpu.sync_copy(qidx_hbm.at[wid], qidx_v)
        pltpu.sync_copy(didx_hbm.at[wid], didx_v)

        def q_issue(c, b):
            start = pl.multiple_of(c * Q_CHUNK_ROWS, 8)
            idx = qidx_v.at[pl.ds(start, Q_CHUNK_ROWS)]
            pltpu.async_copy(table_hbm.at[idx], qbufs[b], qsems[b])

        def d_issue(c, b):
            start = pl.multiple_of(c * D_CHUNK_PAD, 8)
            idx = didx_v.at[pl.ds(start, D_CHUNK_PAD)]
            pltpu.async_copy(table_hbm.at[idx], dbufs[b], dsems[b])

        def reduce_seg(buf, row0, seg_len, stage, out_row):
            half = seg_len // 2

            def load2(r, lanes):
                return plsc.unpack(buf[r, lanes],
                                   format=plsc.PackFormat.INTERLEAVED)

            for j in range(EMB // 32):
                lanes = pl.ds(j * 32, 32)
                a0, a1 = load2(row0, lanes)
                b0, b1 = load2(row0 + half, lanes)
                for r in range(1, half):
                    u0, u1 = load2(row0 + r, lanes)
                    a0, a1 = a0 + u0, a1 + u1
                    v0, v1 = load2(row0 + half + r, lanes)
                    b0, b1 = b0 + v0, b1 + v1
                stage[out_row, pl.ds(j * 32, 16)] = a0 + b0
                stage[out_row, pl.ds(j * 32 + 16, 16)] = a1 + b1

        for b in range(NBUF_Q):
            q_issue(b, b)

        def q_group(g, carry):
            for b in range(NBUF_Q):
                c = g * NBUF_Q + b
                pltpu.make_async_copy(
                    table_hbm.at[qidx_v.at[pl.ds(0, Q_CHUNK_ROWS)]],
                    qbufs[b], qsems[b]).wait()
                for s in range(Q_SEGS_CHUNK):
                    reduce_seg(qbufs[b], s * QLEN, QLEN, qstage_v,
                               c * Q_SEGS_CHUNK + s)
                nxt = c + NBUF_Q
                @pl.when(nxt < Q_CHUNKS)
                def _():
                    q_issue(nxt, b)
            return carry
        lax.fori_loop(0, Q_GROUPS, q_group, 0)
        pltpu.sync_copy(qstage_v, qsum_hbm.at[pl.ds(wid * Q_SEGS_W, Q_SEGS_W)])

        for b in range(NBUF_D):
            d_issue(b, b)

        def d_block(blk, carry):
            def d_group(g, inner):
                for b in range(NBUF_D):
                    local = g * NBUF_D + b
                    c = blk * D_FLUSH_CHUNKS + local
                    pltpu.make_async_copy(
                        table_hbm.at[didx_v.at[pl.ds(0, D_CHUNK_PAD)]],
                        dbufs[b], dsems[b]).wait()
                    for s in range(D_SEGS_CHUNK):
                        reduce_seg(dbufs[b], s * DLEN, DLEN, dstage_v,
                                   local * D_SEGS_CHUNK + s)
                    nxt = c + NBUF_D
                    @pl.when(nxt < D_CHUNKS)
                    def _():
                        d_issue(nxt, b)
                return inner
            lax.fori_loop(0, D_GROUPS, d_group, 0)
            nsegs = D_SEGS_CHUNK * D_FLUSH_CHUNKS
            pltpu.sync_copy(
                dstage_v,
                dsum_hbm.at[pl.ds(wid * D_SEGS_W + blk * nsegs, nsegs)])
            return carry
        lax.fori_loop(0, D_BLOCKS, d_block, 0)

    return pool(table, qidx, didx)


def _cosine_tc(qsum, dsum):
    BLK = 256

    def body(q_ref, d_ref, o_ref):
        q = q_ref[...] * (1.0 / QLEN)
        qn = jnp.maximum(
            jnp.sqrt(jnp.sum(q * q, axis=1, keepdims=True)), 1e-8)
        cols = []
        for dd in range(NDOCS):
            dj = d_ref[:, dd * EMB:(dd + 1) * EMB] * (1.0 / DLEN)
            dot = jnp.sum(q * dj, axis=1, keepdims=True)
            dn = jnp.maximum(
                jnp.sqrt(jnp.sum(dj * dj, axis=1, keepdims=True)), 1e-8)
            cols.append(dot / (qn * dn))
        o_ref[...] = jnp.concatenate(cols, axis=1)

    return pl.pallas_call(
        body,
        grid=(B // BLK,),
        in_specs=[
            pl.BlockSpec((BLK, EMB), lambda i: (i, 0)),
            pl.BlockSpec((BLK, NDOCS * EMB), lambda i: (i, 0)),
        ],
        out_specs=pl.BlockSpec((BLK, NDOCS), lambda i: (i, 0)),
        out_shape=jax.ShapeDtypeStruct((B, NDOCS), jnp.float32),
    )(qsum, dsum)


def kernel(batch_queries, query_len, batch_docs, doc_len, W):
    del query_len, doc_len
    W = W.astype(jnp.bfloat16)
    qidx = batch_queries.astype(jnp.int32).reshape(NW, Q_SEGS_W * QLEN)
    d = batch_docs.astype(jnp.int32).reshape(NW, D_CHUNKS, D_CHUNK_ROWS_REAL)
    pad = jnp.zeros((NW, D_CHUNKS, D_CHUNK_PAD - D_CHUNK_ROWS_REAL),
                    jnp.int32)
    didx = jnp.concatenate([d, pad], axis=2).reshape(NW, -1)
    qsum, dsum = _sc_pool(W, qidx, didx)
    scores = _cosine_tc(qsum, dsum.reshape(B, NDOCS * EMB))
    return scores


# speedup vs baseline: 45.4228x; 2.6112x over previous
import functools

import jax
import jax.numpy as jnp
from jax import lax
from jax.experimental import pallas as pl
from jax.experimental.pallas import tpu as pltpu
from jax.experimental.pallas import tpu_sc as plsc

VOCAB_ROWS = 100000
EMB = 64
B = 4096
QLEN = 20
NDOCS = 8
DLEN = 50

NC = 2
NS = 16
NW = NC * NS

Q_SEGS_W = B // NW
Q_SEGS_CHUNK = 4
Q_CHUNK_ROWS = Q_SEGS_CHUNK * QLEN
Q_CHUNKS = Q_SEGS_W // Q_SEGS_CHUNK

D_SEGS = B * NDOCS
D_SEGS_W = D_SEGS // NW
D_SEGS_CHUNK = 4
D_CHUNK_ROWS = D_SEGS_CHUNK * DLEN
D_SPLIT = 104
D_CHUNKS = D_SEGS_W // D_SEGS_CHUNK
D_FLUSH_CHUNKS = 32
D_BLOCKS = D_CHUNKS // D_FLUSH_CHUNKS


def _sc_pool(table, qidx, didx):
    mesh = plsc.VectorSubcoreMesh(
        core_axis_name="c", subcore_axis_name="s",
        num_cores=NC, num_subcores=NS)

    NBUF_D = 2
    NBUF_Q = 2
    D_GROUPS = D_FLUSH_CHUNKS // NBUF_D
    Q_GROUPS = Q_CHUNKS // NBUF_Q

    @functools.partial(
        pl.kernel,
        out_type=[
            jax.ShapeDtypeStruct((B, EMB), jnp.float32),
            jax.ShapeDtypeStruct((D_SEGS, EMB), jnp.float32),
        ],
        mesh=mesh,
        compiler_params=pltpu.CompilerParams(
            use_tc_tiling_on_sc=False, needs_layout_passes=False),
        scratch_types=[
            pltpu.VMEM((Q_SEGS_W * QLEN,), jnp.int32),
            pltpu.VMEM((D_SEGS_W * DLEN,), jnp.int32),
            [pltpu.VMEM((Q_CHUNK_ROWS, EMB), jnp.bfloat16)] * NBUF_Q,
            [pltpu.VMEM((D_CHUNK_ROWS, EMB), jnp.bfloat16)] * NBUF_D,
            pltpu.VMEM((Q_SEGS_W, EMB), jnp.float32),
            pltpu.VMEM((D_SEGS_CHUNK * D_FLUSH_CHUNKS, EMB), jnp.float32),
            [pltpu.SemaphoreType.DMA] * NBUF_Q,
            [pltpu.SemaphoreType.DMA] * NBUF_D,
        ],
    )
    def pool(table_hbm, qidx_hbm, didx_hbm, qsum_hbm, dsum_hbm,
             qidx_v, didx_v, qbufs, dbufs, qstage_v, dstage_v, qsems, dsems):
        wid = lax.axis_index("s") * NC + lax.axis_index("c")

        pltpu.sync_copy(qidx_hbm.at[wid], qidx_v)
        pltpu.sync_copy(didx_hbm.at[wid], didx_v)

        def q_issue(c, b):
            start = pl.multiple_of(c * Q_CHUNK_ROWS, 8)
            idx = qidx_v.at[pl.ds(start, Q_CHUNK_ROWS)]
            pltpu.async_copy(table_hbm.at[idx], qbufs[b], qsems[b])

        def d_issue(c, b):
            start = pl.multiple_of(c * D_CHUNK_ROWS, 8)
            idx0 = didx_v.at[pl.ds(start, D_SPLIT)]
            start1 = pl.multiple_of(c * D_CHUNK_ROWS + D_SPLIT, 8)
            idx1 = didx_v.at[pl.ds(start1, D_CHUNK_ROWS - D_SPLIT)]
            pltpu.async_copy(table_hbm.at[idx0],
                             dbufs[b].at[pl.ds(0, D_SPLIT)], dsems[b])
            pltpu.async_copy(table_hbm.at[idx1],
                             dbufs[b].at[pl.ds(D_SPLIT,
                                               D_CHUNK_ROWS - D_SPLIT)],
                             dsems[b])

        def d_wait(b):
            pltpu.make_async_copy(
                table_hbm.at[didx_v.at[pl.ds(0, D_SPLIT)]],
                dbufs[b].at[pl.ds(0, D_SPLIT)], dsems[b]).wait()
            pltpu.make_async_copy(
                table_hbm.at[didx_v.at[pl.ds(0, D_CHUNK_ROWS - D_SPLIT)]],
                dbufs[b].at[pl.ds(D_SPLIT, D_CHUNK_ROWS - D_SPLIT)],
                dsems[b]).wait()

        def reduce_seg(buf, row0, seg_len, stage, out_row):
            half = seg_len // 2

            def load2(r, lanes):
                return plsc.unpack(buf[r, lanes],
                                   format=plsc.PackFormat.INTERLEAVED)

            for j in range(EMB // 32):
                lanes = pl.ds(j * 32, 32)
                a0, a1 = load2(row0, lanes)
                b0, b1 = load2(row0 + half, lanes)
                for r in range(1, half):
                    u0, u1 = load2(row0 + r, lanes)
                    a0, a1 = a0 + u0, a1 + u1
                    v0, v1 = load2(row0 + half + r, lanes)
                    b0, b1 = b0 + v0, b1 + v1
                stage[out_row, pl.ds(j * 32, 16)] = a0 + b0
                stage[out_row, pl.ds(j * 32 + 16, 16)] = a1 + b1

        for b in range(NBUF_Q):
            q_issue(b, b)

        def q_group(g, carry):
            for b in range(NBUF_Q):
                c = g * NBUF_Q + b
                pltpu.make_async_copy(
                    table_hbm.at[qidx_v.at[pl.ds(0, Q_CHUNK_ROWS)]],
                    qbufs[b], qsems[b]).wait()
                for s in range(Q_SEGS_CHUNK):
                    reduce_seg(qbufs[b], s * QLEN, QLEN, qstage_v,
                               c * Q_SEGS_CHUNK + s)
                nxt = c + NBUF_Q
                @pl.when(nxt < Q_CHUNKS)
                def _():
                    q_issue(nxt, b)
            return carry
        lax.fori_loop(0, Q_GROUPS, q_group, 0)
        pltpu.sync_copy(qstage_v, qsum_hbm.at[pl.ds(wid * Q_SEGS_W, Q_SEGS_W)])

        for b in range(NBUF_D):
            d_issue(b, b)

        def d_block(blk, carry):
            def d_group(g, inner):
                for b in range(NBUF_D):
                    local = g * NBUF_D + b
                    c = blk * D_FLUSH_CHUNKS + local
                    d_wait(b)
                    for s in range(D_SEGS_CHUNK):
                        reduce_seg(dbufs[b], s * DLEN, DLEN, dstage_v,
                                   local * D_SEGS_CHUNK + s)
                    nxt = c + NBUF_D
                    @pl.when(nxt < D_CHUNKS)
                    def _():
                        d_issue(nxt, b)
                return inner
            lax.fori_loop(0, D_GROUPS, d_group, 0)
            nsegs = D_SEGS_CHUNK * D_FLUSH_CHUNKS
            pltpu.sync_copy(
                dstage_v,
                dsum_hbm.at[pl.ds(wid * D_SEGS_W + blk * nsegs, nsegs)])
            return carry
        lax.fori_loop(0, D_BLOCKS, d_block, 0)

    return pool(table, qidx, didx)


def _cosine_tc(qsum, dsum):
    BLK = 256

    def body(q_ref, d_ref, o_ref):
        q = q_ref[...] * (1.0 / QLEN)
        qn = jnp.maximum(
            jnp.sqrt(jnp.sum(q * q, axis=1, keepdims=True)), 1e-8)
        cols = []
        for dd in range(NDOCS):
            dj = d_ref[:, dd * EMB:(dd + 1) * EMB] * (1.0 / DLEN)
            dot = jnp.sum(q * dj, axis=1, keepdims=True)
            dn = jnp.maximum(
                jnp.sqrt(jnp.sum(dj * dj, axis=1, keepdims=True)), 1e-8)
            cols.append(dot / (qn * dn))
        o_ref[...] = jnp.concatenate(cols, axis=1)

    return pl.pallas_call(
        body,
        grid=(B // BLK,),
        in_specs=[
            pl.BlockSpec((BLK, EMB), lambda i: (i, 0)),
            pl.BlockSpec((BLK, NDOCS * EMB), lambda i: (i, 0)),
        ],
        out_specs=pl.BlockSpec((BLK, NDOCS), lambda i: (i, 0)),
        out_shape=jax.ShapeDtypeStruct((B, NDOCS), jnp.float32),
    )(qsum, dsum)


def kernel(batch_queries, query_len, batch_docs, doc_len, W):
    del query_len, doc_len
    W = W.astype(jnp.bfloat16)
    qidx = batch_queries.astype(jnp.int32).reshape(NW, Q_SEGS_W * QLEN)
    didx = batch_docs.astype(jnp.int32).reshape(NW, D_SEGS_W * DLEN)
    qsum, dsum = _sc_pool(W, qidx, didx)
    scores = _cosine_tc(qsum, dsum.reshape(B, NDOCS * EMB))
    return scores


# speedup vs baseline: 52.8705x; 1.1640x over previous
import functools

import jax
import jax.numpy as jnp
from jax import lax
from jax.experimental import pallas as pl
from jax.experimental.pallas import tpu as pltpu
from jax.experimental.pallas import tpu_sc as plsc

VOCAB_ROWS = 100000
EMB = 64
B = 4096
QLEN = 20
NDOCS = 8
DLEN = 50
EPS = 1e-8

NC = 2
NS = 16
NW = NC * NS

Q_SEGS_W = B // NW
Q_SEGS_CHUNK = 4
Q_CHUNK_ROWS = Q_SEGS_CHUNK * QLEN
Q_CHUNKS = Q_SEGS_W // Q_SEGS_CHUNK

D_SEGS = B * NDOCS
D_SEGS_W = D_SEGS // NW
D_SEGS_CHUNK = 4
D_CHUNK_ROWS = D_SEGS_CHUNK * DLEN
D_SPLIT = 104
D_CHUNKS = D_SEGS_W // D_SEGS_CHUNK


def _rsqrt(x):
    y = plsc.bitcast(
        jnp.full((16,), 0x5F3759DF, jnp.int32)
        - lax.shift_right_logical(plsc.bitcast(x, jnp.int32), 1),
        jnp.float32)
    xh = x * 0.5
    for _ in range(4):
        y = y * (1.5 - xh * y * y)
    return y


def _sc_scores(table, qidx, didx):
    mesh = plsc.VectorSubcoreMesh(
        core_axis_name="c", subcore_axis_name="s",
        num_cores=NC, num_subcores=NS)

    NBUF_D = 2
    NBUF_Q = 2
    D_GROUPS = D_CHUNKS // NBUF_D
    Q_GROUPS = Q_CHUNKS // NBUF_Q

    @functools.partial(
        pl.kernel,
        out_type=jax.ShapeDtypeStruct((B * NDOCS,), jnp.float32),
        mesh=mesh,
        compiler_params=pltpu.CompilerParams(
            use_tc_tiling_on_sc=False, needs_layout_passes=False),
        scratch_types=[
            pltpu.VMEM((Q_SEGS_W * QLEN,), jnp.int32),
            pltpu.VMEM((D_SEGS_W * DLEN,), jnp.int32),
            [pltpu.VMEM((Q_CHUNK_ROWS, EMB), jnp.bfloat16)] * NBUF_Q,
            [pltpu.VMEM((D_CHUNK_ROWS, EMB), jnp.bfloat16)] * NBUF_D,
            pltpu.VMEM((Q_SEGS_W, EMB), jnp.float32),
            pltpu.VMEM((Q_SEGS_W * NDOCS,), jnp.float32),
            [pltpu.SemaphoreType.DMA] * NBUF_Q,
            [pltpu.SemaphoreType.DMA] * NBUF_D,
        ],
    )
    def pool(table_hbm, qidx_hbm, didx_hbm, scores_hbm,
             qidx_v, didx_v, qbufs, dbufs, qstage_v, sstage_v, qsems, dsems):
        wid = lax.axis_index("s") * NC + lax.axis_index("c")

        pltpu.sync_copy(qidx_hbm.at[wid], qidx_v)
        pltpu.sync_copy(didx_hbm.at[wid], didx_v)

        def q_issue(c, b):
            start = pl.multiple_of(c * Q_CHUNK_ROWS, 8)
            idx = qidx_v.at[pl.ds(start, Q_CHUNK_ROWS)]
            pltpu.async_copy(table_hbm.at[idx], qbufs[b], qsems[b])

        def d_issue(c, b):
            start = pl.multiple_of(c * D_CHUNK_ROWS, 8)
            idx0 = didx_v.at[pl.ds(start, D_SPLIT)]
            start1 = pl.multiple_of(c * D_CHUNK_ROWS + D_SPLIT, 8)
            idx1 = didx_v.at[pl.ds(start1, D_CHUNK_ROWS - D_SPLIT)]
            pltpu.async_copy(table_hbm.at[idx0],
                             dbufs[b].at[pl.ds(0, D_SPLIT)], dsems[b])
            pltpu.async_copy(table_hbm.at[idx1],
                             dbufs[b].at[pl.ds(D_SPLIT,
                                               D_CHUNK_ROWS - D_SPLIT)],
                             dsems[b])

        def d_wait(b):
            pltpu.make_async_copy(
                table_hbm.at[didx_v.at[pl.ds(0, D_SPLIT)]],
                dbufs[b].at[pl.ds(0, D_SPLIT)], dsems[b]).wait()
            pltpu.make_async_copy(
                table_hbm.at[didx_v.at[pl.ds(0, D_CHUNK_ROWS - D_SPLIT)]],
                dbufs[b].at[pl.ds(D_SPLIT, D_CHUNK_ROWS - D_SPLIT)],
                dsems[b]).wait()

        def reduce_seg(buf, row0, seg_len):
            half = seg_len // 2

            def load2(r, lanes):
                return plsc.unpack(buf[r, lanes],
                                   format=plsc.PackFormat.INTERLEAVED)

            sums = []
            for j in range(EMB // 32):
                lanes = pl.ds(j * 32, 32)
                a0, a1 = load2(row0, lanes)
                b0, b1 = load2(row0 + half, lanes)
                for r in range(1, half):
                    u0, u1 = load2(row0 + r, lanes)
                    a0, a1 = a0 + u0, a1 + u1
                    v0, v1 = load2(row0 + half + r, lanes)
                    b0, b1 = b0 + v0, b1 + v1
                sums.append(a0 + b0)
                sums.append(a1 + b1)
            return sums

        for b in range(NBUF_Q):
            q_issue(b, b)

        def q_group(g, carry):
            for b in range(NBUF_Q):
                c = g * NBUF_Q + b
                pltpu.make_async_copy(
                    table_hbm.at[qidx_v.at[pl.ds(0, Q_CHUNK_ROWS)]],
                    qbufs[b], qsems[b]).wait()
                for s in range(Q_SEGS_CHUNK):
                    sums = reduce_seg(qbufs[b], s * QLEN, QLEN)
                    for j in range(4):
                        qstage_v[c * Q_SEGS_CHUNK + s,
                                 pl.ds(j * 16, 16)] = sums[j]
                nxt = c + NBUF_Q
                @pl.when(nxt < Q_CHUNKS)
                def _():
                    q_issue(nxt, b)
            return carry
        lax.fori_loop(0, Q_GROUPS, q_group, 0)

        for b in range(NBUF_D):
            d_issue(b, b)

        lanes16 = lax.broadcasted_iota(jnp.int32, (16,), 0)

        def d_group(q, carry):
            qs = [qstage_v[q, pl.ds(j * 16, 16)] for j in range(4)]
            qn2 = jnp.sum(qs[0] * qs[0] + qs[1] * qs[1]
                          + qs[2] * qs[2] + qs[3] * qs[3])
            qinv = _rsqrt(jnp.maximum(jnp.full((16,), qn2, jnp.float32),
                                      (QLEN * EPS) ** 2))
            dotv = jnp.zeros((16,), jnp.float32)
            dn2v = jnp.zeros((16,), jnp.float32)
            for b in range(NBUF_D):
                c = q * NBUF_D + b
                d_wait(b)
                for s in range(D_SEGS_CHUNK):
                    n = b * D_SEGS_CHUNK + s
                    ds_ = reduce_seg(dbufs[b], s * DLEN, DLEN)
                    dot = jnp.sum(qs[0] * ds_[0] + qs[1] * ds_[1]
                                  + qs[2] * ds_[2] + qs[3] * ds_[3])
                    dn2 = jnp.sum(ds_[0] * ds_[0] + ds_[1] * ds_[1]
                                  + ds_[2] * ds_[2] + ds_[3] * ds_[3])
                    dotv = jnp.where(lanes16 == n, dot, dotv)
                    dn2v = jnp.where(lanes16 == n, dn2, dn2v)
                nxt = c + NBUF_D
                @pl.when(nxt < D_CHUNKS)
                def _():
                    d_issue(nxt, b)
            dinv = _rsqrt(jnp.maximum(dn2v, (DLEN * EPS) ** 2))
            score = dotv * qinv * dinv
            plsc.store_scatter(sstage_v, [q * NDOCS + lanes16], score,
                               mask=lanes16 < NDOCS)
            return carry
        lax.fori_loop(0, D_GROUPS, d_group, 0)

        pltpu.sync_copy(
            sstage_v,
            scores_hbm.at[pl.ds(wid * Q_SEGS_W * NDOCS, Q_SEGS_W * NDOCS)])

    return pool(table, qidx, didx)


def kernel(batch_queries, query_len, batch_docs, doc_len, W):
    del query_len, doc_len
    W = W.astype(jnp.bfloat16)
    qidx = batch_queries.astype(jnp.int32).reshape(NW, Q_SEGS_W * QLEN)
    didx = batch_docs.astype(jnp.int32).reshape(NW, D_SEGS_W * DLEN)
    return _sc_scores(W, qidx, didx).reshape(B, NDOCS)


# speedup vs baseline: 57.5582x; 1.0887x over previous
import functools

import jax
import jax.numpy as jnp
from jax import lax
from jax.experimental import pallas as pl
from jax.experimental.pallas import tpu as pltpu
from jax.experimental.pallas import tpu_sc as plsc

VOCAB_ROWS = 100000
EMB = 64
B = 4096
QLEN = 20
NDOCS = 8
DLEN = 50
EPS = 1e-8

NC = 2
NS = 16
NW = NC * NS

Q_SEGS_W = B // NW

D_SEGS_W = B * NDOCS // NW


def _rsqrt(x):
    y = plsc.bitcast(
        jnp.full((16,), 0x5F3759DF, jnp.int32)
        - lax.shift_right_logical(plsc.bitcast(x, jnp.int32), 1),
        jnp.float32)
    xh = x * 0.5
    for _ in range(4):
        y = y * (1.5 - xh * y * y)
    return y


def _sc_scores(table, qidx, didx):
    mesh = plsc.VectorSubcoreMesh(
        core_axis_name="c", subcore_axis_name="s",
        num_cores=NC, num_subcores=NS)

    NBUF_D = 8
    NBUF_Q = 4
    Q_GROUPS = Q_SEGS_W // NBUF_Q

    @functools.partial(
        pl.kernel,
        out_type=jax.ShapeDtypeStruct((B * NDOCS,), jnp.float32),
        mesh=mesh,
        compiler_params=pltpu.CompilerParams(
            use_tc_tiling_on_sc=False, needs_layout_passes=False),
        scratch_types=[
            pltpu.VMEM((Q_SEGS_W, QLEN), jnp.int32),
            pltpu.VMEM((Q_SEGS_W, NDOCS, DLEN), jnp.int32),
            [pltpu.VMEM((QLEN, EMB), jnp.bfloat16)] * NBUF_Q,
            [pltpu.VMEM((DLEN, EMB), jnp.bfloat16)] * NBUF_D,
            pltpu.VMEM((Q_SEGS_W, EMB), jnp.float32),
            pltpu.VMEM((Q_SEGS_W * NDOCS,), jnp.float32),
            [pltpu.SemaphoreType.DMA] * NBUF_Q,
            [pltpu.SemaphoreType.DMA] * NBUF_D,
        ],
    )
    def pool(table_hbm, qidx_hbm, didx_hbm, scores_hbm,
             qidx_v, didx_v, qbufs, dbufs, qstage_v, sstage_v, qsems, dsems):
        wid = lax.axis_index("s") * NC + lax.axis_index("c")

        pltpu.sync_copy(qidx_hbm.at[pl.ds(wid * Q_SEGS_W, Q_SEGS_W)], qidx_v)
        pltpu.sync_copy(didx_hbm.at[pl.ds(wid * Q_SEGS_W, Q_SEGS_W)], didx_v)

        def q_issue(c, b):
            idx = qidx_v.at[c]
            pltpu.async_copy(table_hbm.at[idx], qbufs[b], qsems[b])

        def q_wait(b):
            pltpu.make_async_copy(
                table_hbm.at[qidx_v.at[0]],
                qbufs[b], qsems[b]).wait()

        def d_issue(c, b):
            idx = didx_v.at[c // NDOCS, c % NDOCS]
            pltpu.async_copy(table_hbm.at[idx], dbufs[b], dsems[b])

        def d_wait(b):
            pltpu.make_async_copy(
                table_hbm.at[didx_v.at[0, 0]],
                dbufs[b], dsems[b]).wait()

        def reduce_seg(buf, row0, seg_len):
            half = seg_len // 2

            def load2(r, lanes):
                return plsc.unpack(buf[r, lanes],
                                   format=plsc.PackFormat.INTERLEAVED)

            sums = []
            for j in range(EMB // 32):
                lanes = pl.ds(j * 32, 32)
                a0, a1 = load2(row0, lanes)
                b0, b1 = load2(row0 + half, lanes)
                for r in range(1, half):
                    u0, u1 = load2(row0 + r, lanes)
                    a0, a1 = a0 + u0, a1 + u1
                    v0, v1 = load2(row0 + half + r, lanes)
                    b0, b1 = b0 + v0, b1 + v1
                sums.append(a0 + b0)
                sums.append(a1 + b1)
            return sums

        for b in range(NBUF_Q):
            q_issue(b, b)

        def q_group(g, carry):
            for b in range(NBUF_Q):
                c = g * NBUF_Q + b
                q_wait(b)
                sums = reduce_seg(qbufs[b], 0, QLEN)
                for j in range(4):
                    qstage_v[c, pl.ds(j * 16, 16)] = sums[j]
                nxt = c + NBUF_Q
                @pl.when(nxt < Q_SEGS_W)
                def _():
                    q_issue(nxt, b)
            return carry
        lax.fori_loop(0, Q_GROUPS, q_group, 0)

        for b in range(NBUF_D):
            d_issue(b, b)

        lanes16 = lax.broadcasted_iota(jnp.int32, (16,), 0)

        def d_group(q, carry):
            qs = [qstage_v[q, pl.ds(j * 16, 16)] for j in range(4)]
            qn2 = jnp.sum(qs[0] * qs[0] + qs[1] * qs[1]
                          + qs[2] * qs[2] + qs[3] * qs[3])
            qinv = _rsqrt(jnp.maximum(jnp.full((16,), qn2, jnp.float32),
                                      (QLEN * EPS) ** 2))
            dotv = jnp.zeros((16,), jnp.float32)
            dn2v = jnp.zeros((16,), jnp.float32)
            for b in range(NBUF_D):
                c = q * NBUF_D + b
                d_wait(b)
                ds_ = reduce_seg(dbufs[b], 0, DLEN)
                dot = jnp.sum(qs[0] * ds_[0] + qs[1] * ds_[1]
                              + qs[2] * ds_[2] + qs[3] * ds_[3])
                dn2 = jnp.sum(ds_[0] * ds_[0] + ds_[1] * ds_[1]
                              + ds_[2] * ds_[2] + ds_[3] * ds_[3])
                dotv = jnp.where(lanes16 == b, dot, dotv)
                dn2v = jnp.where(lanes16 == b, dn2, dn2v)
                nxt = c + NBUF_D
                @pl.when(nxt < D_SEGS_W)
                def _():
                    d_issue(nxt, b)
            dinv = _rsqrt(jnp.maximum(dn2v, (DLEN * EPS) ** 2))
            score = dotv * qinv * dinv
            plsc.store_scatter(sstage_v, [q * NDOCS + lanes16], score,
                               mask=lanes16 < NDOCS)
            return carry
        lax.fori_loop(0, Q_SEGS_W, d_group, 0)

        pltpu.sync_copy(
            sstage_v,
            scores_hbm.at[pl.ds(wid * Q_SEGS_W * NDOCS, Q_SEGS_W * NDOCS)])

    return pool(table, qidx, didx)


def kernel(batch_queries, query_len, batch_docs, doc_len, W):
    del query_len, doc_len
    W = W.astype(jnp.bfloat16)
    qidx = batch_queries.astype(jnp.int32)
    didx = batch_docs.astype(jnp.int32)
    return _sc_scores(W, qidx, didx).reshape(B, NDOCS)
